# Initial kernel scaffold; baseline (speedup 1.0000x reference)
import jax
import jax.numpy as jnp
from jax.experimental import pallas as pl


def kernel(x, norm_w, Wg_router, gate_bias, Ws_gate, Ws_up, Ws_down, We_gate, We_up, We_down):
    raise NotImplementedError("write your pallas kernel here")


# speedup vs baseline: 1.4393x; 1.4393x over previous
import functools

import jax
import jax.numpy as jnp
from jax import lax
from jax.experimental import pallas as pl
from jax.experimental.pallas import tpu as pltpu
from jax.experimental.pallas import tpu_sc as plsc

T, D, E, K, IM, IS = 2048, 1024, 8, 2, 512, 512
BT = 512
NB = 16
NSLOT = NB * BT
NBD = NB + 1
NW = 32
TPW = T // NW


def _silu(v):
    return v * jax.nn.sigmoid(v)


def _router_body(x_ref, nw_ref, wg_ref, gb_ref, h_ref, meta_ref, smeta_ref):
    x = x_ref[...]
    h = x * lax.rsqrt(jnp.mean(x * x, axis=-1, keepdims=True) + 1e-6)
    h = h * nw_ref[...]
    h_ref[...] = h

    logits = lax.dot_general(h, wg_ref[...], (((1,), (1,)), ((), ())),
                             preferred_element_type=jnp.float32)
    scores = jax.nn.sigmoid(logits)
    biased = scores + gb_ref[...]

    eids = lax.broadcasted_iota(jnp.int32, (T, E), 1)
    m1 = jnp.max(biased, axis=1, keepdims=True)
    i1 = jnp.min(jnp.where(biased == m1, eids, E), axis=1, keepdims=True)
    b2 = jnp.where(eids == i1, -jnp.inf, biased)
    m2 = jnp.max(b2, axis=1, keepdims=True)
    i2 = jnp.min(jnp.where(b2 == m2, eids, E), axis=1, keepdims=True)

    sel1 = (eids == i1)
    sel2 = (eids == i2)
    w1 = jnp.sum(jnp.where(sel1, scores, 0.0), axis=1, keepdims=True)
    w2 = jnp.sum(jnp.where(sel2, scores, 0.0), axis=1, keepdims=True)
    den = w1 + w2 + 1e-20
    w1 = w1 / den
    w2 = w2 / den

    onehot = (sel1 | sel2).astype(jnp.bfloat16)
    rows = lax.broadcasted_iota(jnp.int32, (T, T), 0)
    cols = lax.broadcasted_iota(jnp.int32, (T, T), 1)
    lt = (cols < rows).astype(jnp.bfloat16)
    rank = lax.dot_general(lt, onehot, (((1,), (0,)), ((), ())),
                           preferred_element_type=jnp.float32)

    counts = jnp.sum(onehot.astype(jnp.float32), axis=0, keepdims=True)
    pb = jnp.floor((counts + (BT - 1)) * (1.0 / BT))
    lt8 = (lax.broadcasted_iota(jnp.int32, (E, E), 0)
           < lax.broadcasted_iota(jnp.int32, (E, E), 1)).astype(jnp.float32)
    offb = lax.dot_general(pb, lt8, (((1,), (0,)), ((), ())),
                           preferred_element_type=jnp.float32)
    nb_used = jnp.sum(pb, axis=1, keepdims=True)

    offs = offb * float(BT)
    off1 = jnp.sum(jnp.where(sel1, offs, 0.0), axis=1, keepdims=True)
    off2 = jnp.sum(jnp.where(sel2, offs, 0.0), axis=1, keepdims=True)
    r1 = jnp.sum(jnp.where(sel1, rank, 0.0), axis=1, keepdims=True)
    r2 = jnp.sum(jnp.where(sel2, rank, 0.0), axis=1, keepdims=True)
    p1 = off1 + r1
    p2 = off2 + r2

    meta_ref[...] = lax.concatenate(
        [p1, p2, w1, w2, jnp.zeros((T, 4), jnp.float32)], 1)
    srow = lax.concatenate([offb, nb_used, jnp.zeros((1, 7), jnp.float32)], 1)
    smeta_ref[...] = jnp.broadcast_to(srow, (8, 16)).astype(jnp.int32)


def _router_call(x, norm_w, Wg_router, gate_bias):
    return pl.pallas_call(
        _router_body,
        out_shape=(
            jax.ShapeDtypeStruct((T, D), jnp.float32),
            jax.ShapeDtypeStruct((T, 8), jnp.float32),
            jax.ShapeDtypeStruct((8, 16), jnp.int32),
        ),
    )(x, norm_w.reshape(1, D), Wg_router, gate_bias.reshape(1, E))


def _dispatch_sc(h, p1, p2):
    mesh = plsc.VectorSubcoreMesh(core_axis_name="c", subcore_axis_name="s")

    @functools.partial(
        pl.kernel, mesh=mesh,
        out_type=jax.ShapeDtypeStruct((NSLOT, D), jnp.float32),
        scratch_types=[
            pltpu.VMEM((TPW,), jnp.int32),
            pltpu.VMEM((TPW,), jnp.int32),
            pltpu.VMEM((TPW, D), jnp.float32),
            pltpu.SemaphoreType.DMA,
        ],
    )
    def body(h_hbm, p1_hbm, p2_hbm, hs_hbm, idx1_v, idx2_v, rows_v, sem):
        wid = lax.axis_index("s") * 2 + lax.axis_index("c")
        base = wid * TPW
        pltpu.sync_copy(p1_hbm.at[pl.ds(base, TPW)], idx1_v)
        pltpu.sync_copy(p2_hbm.at[pl.ds(base, TPW)], idx2_v)
        pltpu.sync_copy(h_hbm.at[pl.ds(base, TPW)], rows_v)
        pltpu.async_copy(rows_v, hs_hbm.at[idx1_v], sem).wait()
        pltpu.async_copy(rows_v, hs_hbm.at[idx2_v], sem).wait()

    return body(h, p1, p2)


def _ffn_body(smeta_ref, hs_ref, wg_ref, wu_ref, wd_ref, ys_ref):
    b = pl.program_id(0)

    @pl.when(b < smeta_ref[8])
    def _():
        hb = hs_ref[...]
        g = jnp.dot(hb, wg_ref[0], preferred_element_type=jnp.float32)
        u = jnp.dot(hb, wu_ref[0], preferred_element_type=jnp.float32)
        act = _silu(g) * u
        ys_ref[...] = jnp.dot(act, wd_ref[0], preferred_element_type=jnp.float32)


def _block_expert(bc, sm):
    be = jnp.int32(-1)
    for e in range(E):
        be = be + jnp.where(sm[e] <= bc, 1, 0).astype(jnp.int32)
    return be


def _ffn_call(smeta, hs, We_gate, We_up, We_down):
    def hs_map(b, sm):
        return (jnp.minimum(b, sm[8]), 0)

    def w_map(b, sm):
        return (_block_expert(jnp.minimum(b, sm[8] - 1), sm), 0, 0)

    def ys_map(b, sm):
        return (jnp.minimum(b, sm[8]), 0)

    spec = pltpu.PrefetchScalarGridSpec(
        num_scalar_prefetch=1,
        grid=(NB,),
        in_specs=[
            pl.BlockSpec((BT, D), hs_map),
            pl.BlockSpec((1, D, IM), w_map),
            pl.BlockSpec((1, D, IM), w_map),
            pl.BlockSpec((1, IM, D), w_map),
        ],
        out_specs=pl.BlockSpec((BT, D), ys_map),
    )
    return pl.pallas_call(
        _ffn_body,
        grid_spec=spec,
        out_shape=jax.ShapeDtypeStruct((NBD * BT, D), jnp.float32),
    )(smeta, hs, We_gate, We_up, We_down)


def _gather_sc(ys, p1, p2):
    mesh = plsc.VectorSubcoreMesh(core_axis_name="c", subcore_axis_name="s")

    @functools.partial(
        pl.kernel, mesh=mesh,
        out_type=(
            jax.ShapeDtypeStruct((T, D), jnp.float32),
            jax.ShapeDtypeStruct((T, D), jnp.float32),
        ),
        scratch_types=[
            pltpu.VMEM((TPW,), jnp.int32),
            pltpu.VMEM((TPW, D), jnp.float32),
            pltpu.SemaphoreType.DMA,
        ],
    )
    def body(ys_hbm, p1_hbm, p2_hbm, g1_hbm, g2_hbm, idx_v, rows_v, sem):
        wid = lax.axis_index("s") * 2 + lax.axis_index("c")
        base = wid * TPW
        pltpu.sync_copy(p1_hbm.at[pl.ds(base, TPW)], idx_v)
        pltpu.async_copy(ys_hbm.at[idx_v], rows_v, sem).wait()
        pltpu.sync_copy(rows_v, g1_hbm.at[pl.ds(base, TPW)])
        pltpu.sync_copy(p2_hbm.at[pl.ds(base, TPW)], idx_v)
        pltpu.async_copy(ys_hbm.at[idx_v], rows_v, sem).wait()
        pltpu.sync_copy(rows_v, g2_hbm.at[pl.ds(base, TPW)])

    return body(ys, p1, p2)


def _combine_body(x_ref, h_ref, meta_ref, g1_ref, g2_ref,
                  wsg_ref, wsu_ref, wsd_ref, out_ref):
    hb = h_ref[...]
    a = jnp.dot(hb, wsg_ref[...], preferred_element_type=jnp.float32)
    u = jnp.dot(hb, wsu_ref[...], preferred_element_type=jnp.float32)
    ysh = jnp.dot(_silu(a) * u, wsd_ref[...], preferred_element_type=jnp.float32)
    w1 = meta_ref[:, 2:3]
    w2 = meta_ref[:, 3:4]
    out_ref[...] = x_ref[...] + ysh + w1 * g1_ref[...] + w2 * g2_ref[...]


def _combine_call(x, h, meta, g1, g2, Ws_gate, Ws_up, Ws_down):
    BTOK = 256
    grid = (T // BTOK,)
    tok = lambda b: (b, 0)
    fixed = lambda b: (0, 0)
    return pl.pallas_call(
        _combine_body,
        grid=grid,
        in_specs=[
            pl.BlockSpec((BTOK, D), tok),
            pl.BlockSpec((BTOK, D), tok),
            pl.BlockSpec((BTOK, 8), tok),
            pl.BlockSpec((BTOK, D), tok),
            pl.BlockSpec((BTOK, D), tok),
            pl.BlockSpec((D, IS), fixed),
            pl.BlockSpec((D, IS), fixed),
            pl.BlockSpec((IS, D), fixed),
        ],
        out_specs=pl.BlockSpec((BTOK, D), tok),
        out_shape=jax.ShapeDtypeStruct((T, D), jnp.float32),
    )(x, h, meta, g1, g2, Ws_gate, Ws_up, Ws_down)


def kernel(x, norm_w, Wg_router, gate_bias, Ws_gate, Ws_up, Ws_down,
           We_gate, We_up, We_down):
    h, meta, smeta8 = _router_call(x, norm_w, Wg_router, gate_bias)
    p1 = meta[:, 0].astype(jnp.int32)
    p2 = meta[:, 1].astype(jnp.int32)
    smeta = smeta8[0]
    hs = _dispatch_sc(h, p1, p2)
    ys = _ffn_call(smeta, hs, We_gate, We_up, We_down)
    g1, g2 = _gather_sc(ys, p1, p2)
    return _combine_call(x, h, meta, g1, g2, Ws_gate, Ws_up, Ws_down)


# speedup vs baseline: 1.4403x; 1.0007x over previous
import functools

import jax
import jax.numpy as jnp
from jax import lax
from jax.experimental import pallas as pl
from jax.experimental.pallas import tpu as pltpu
from jax.experimental.pallas import tpu_sc as plsc

T, D, E, K, IM, IS = 2048, 1024, 8, 2, 512, 512
BT = 512
NB = 16
NSLOT = NB * BT
NBD = NB + 1
NW = 32
TPW = T // NW


def _silu(v):
    return v * jax.nn.sigmoid(v)


def _router_body(x_ref, nw_ref, wg_ref, gb_ref, h_ref, meta_ref, smeta_ref):
    x = x_ref[...]
    h = x * lax.rsqrt(jnp.mean(x * x, axis=-1, keepdims=True) + 1e-6)
    h = h * nw_ref[...]
    h_ref[...] = h

    logits = lax.dot_general(h, wg_ref[...], (((1,), (1,)), ((), ())),
                             preferred_element_type=jnp.float32)
    scores = jax.nn.sigmoid(logits)
    biased = scores + gb_ref[...]

    eids = lax.broadcasted_iota(jnp.int32, (T, E), 1)
    m1 = jnp.max(biased, axis=1, keepdims=True)
    i1 = jnp.min(jnp.where(biased == m1, eids, E), axis=1, keepdims=True)
    b2 = jnp.where(eids == i1, -jnp.inf, biased)
    m2 = jnp.max(b2, axis=1, keepdims=True)
    i2 = jnp.min(jnp.where(b2 == m2, eids, E), axis=1, keepdims=True)

    sel1 = (eids == i1)
    sel2 = (eids == i2)
    w1 = jnp.sum(jnp.where(sel1, scores, 0.0), axis=1, keepdims=True)
    w2 = jnp.sum(jnp.where(sel2, scores, 0.0), axis=1, keepdims=True)
    den = w1 + w2 + 1e-20
    w1 = w1 / den
    w2 = w2 / den

    onehot = (sel1 | sel2).astype(jnp.bfloat16)
    rows = lax.broadcasted_iota(jnp.int32, (T, T), 0)
    cols = lax.broadcasted_iota(jnp.int32, (T, T), 1)
    lt = (cols < rows).astype(jnp.bfloat16)
    rank = lax.dot_general(lt, onehot, (((1,), (0,)), ((), ())),
                           preferred_element_type=jnp.float32)

    counts = jnp.sum(onehot.astype(jnp.float32), axis=0, keepdims=True)
    pb = jnp.floor((counts + (BT - 1)) * (1.0 / BT))
    lt8 = (lax.broadcasted_iota(jnp.int32, (E, E), 0)
           < lax.broadcasted_iota(jnp.int32, (E, E), 1)).astype(jnp.float32)
    offb = lax.dot_general(pb, lt8, (((1,), (0,)), ((), ())),
                           preferred_element_type=jnp.float32)
    nb_used = jnp.sum(pb, axis=1, keepdims=True)

    offs = offb * float(BT)
    off1 = jnp.sum(jnp.where(sel1, offs, 0.0), axis=1, keepdims=True)
    off2 = jnp.sum(jnp.where(sel2, offs, 0.0), axis=1, keepdims=True)
    r1 = jnp.sum(jnp.where(sel1, rank, 0.0), axis=1, keepdims=True)
    r2 = jnp.sum(jnp.where(sel2, rank, 0.0), axis=1, keepdims=True)
    p1 = off1 + r1
    p2 = off2 + r2

    meta_ref[...] = lax.concatenate(
        [p1, p2, w1, w2, jnp.zeros((T, 4), jnp.float32)], 1)
    srow = lax.concatenate([offb, nb_used, jnp.zeros((1, 7), jnp.float32)], 1)
    smeta_ref[...] = jnp.broadcast_to(srow, (8, 16)).astype(jnp.int32)


def _router_call(x, norm_w, Wg_router, gate_bias):
    return pl.pallas_call(
        _router_body,
        out_shape=(
            jax.ShapeDtypeStruct((T, D), jnp.float32),
            jax.ShapeDtypeStruct((T, 8), jnp.float32),
            jax.ShapeDtypeStruct((8, 16), jnp.int32),
        ),
    )(x, norm_w.reshape(1, D), Wg_router, gate_bias.reshape(1, E))


def _dispatch_sc(h, p1, p2):
    mesh = plsc.VectorSubcoreMesh(core_axis_name="c", subcore_axis_name="s")

    @functools.partial(
        pl.kernel, mesh=mesh,
        out_type=jax.ShapeDtypeStruct((NSLOT, D), jnp.float32),
        scratch_types=[
            pltpu.VMEM((TPW,), jnp.int32),
            pltpu.VMEM((TPW,), jnp.int32),
            pltpu.VMEM((TPW, D), jnp.float32),
            pltpu.SemaphoreType.DMA,
        ],
    )
    def body(h_hbm, p1_hbm, p2_hbm, hs_hbm, idx1_v, idx2_v, rows_v, sem):
        wid = lax.axis_index("s") * 2 + lax.axis_index("c")
        base = wid * TPW
        pltpu.sync_copy(p1_hbm.at[pl.ds(base, TPW)], idx1_v)
        pltpu.sync_copy(p2_hbm.at[pl.ds(base, TPW)], idx2_v)
        pltpu.sync_copy(h_hbm.at[pl.ds(base, TPW)], rows_v)
        pltpu.async_copy(rows_v, hs_hbm.at[idx1_v], sem).wait()
        pltpu.async_copy(rows_v, hs_hbm.at[idx2_v], sem).wait()

    return body(h, p1, p2)


def _ffn_body(smeta_ref, hs_ref, wg_ref, wu_ref, wd_ref, ys_ref):
    b = pl.program_id(0)

    @pl.when(b < smeta_ref[8])
    def _():
        hb = hs_ref[...]
        g = jnp.dot(hb, wg_ref[0], preferred_element_type=jnp.float32)
        u = jnp.dot(hb, wu_ref[0], preferred_element_type=jnp.float32)
        act = _silu(g) * u
        ys_ref[...] = jnp.dot(act, wd_ref[0], preferred_element_type=jnp.float32)


def _block_expert(bc, sm):
    be = jnp.int32(-1)
    for e in range(E):
        be = be + jnp.where(sm[e] <= bc, 1, 0).astype(jnp.int32)
    return be


def _ffn_call(smeta, hs, We_gate, We_up, We_down):
    def hs_map(b, sm):
        return (jnp.minimum(b, sm[8]), 0)

    def w_map(b, sm):
        return (_block_expert(jnp.minimum(b, sm[8] - 1), sm), 0, 0)

    def ys_map(b, sm):
        return (jnp.minimum(b, sm[8]), 0)

    spec = pltpu.PrefetchScalarGridSpec(
        num_scalar_prefetch=1,
        grid=(NB,),
        in_specs=[
            pl.BlockSpec((BT, D), hs_map),
            pl.BlockSpec((1, D, IM), w_map),
            pl.BlockSpec((1, D, IM), w_map),
            pl.BlockSpec((1, IM, D), w_map),
        ],
        out_specs=pl.BlockSpec((BT, D), ys_map),
    )
    return pl.pallas_call(
        _ffn_body,
        grid_spec=spec,
        out_shape=jax.ShapeDtypeStruct((NBD * BT, D), jnp.float32),
    )(smeta, hs, We_gate, We_up, We_down)


def _gather_sc(ys, p1, p2):
    mesh = plsc.VectorSubcoreMesh(core_axis_name="c", subcore_axis_name="s")

    @functools.partial(
        pl.kernel, mesh=mesh,
        out_type=(
            jax.ShapeDtypeStruct((T, D), jnp.float32),
            jax.ShapeDtypeStruct((T, D), jnp.float32),
        ),
        scratch_types=[
            pltpu.VMEM((TPW,), jnp.int32),
            pltpu.VMEM((TPW, D), jnp.float32),
            pltpu.SemaphoreType.DMA,
        ],
    )
    def body(ys_hbm, p1_hbm, p2_hbm, g1_hbm, g2_hbm, idx_v, rows_v, sem):
        wid = lax.axis_index("s") * 2 + lax.axis_index("c")
        base = wid * TPW
        pltpu.sync_copy(p1_hbm.at[pl.ds(base, TPW)], idx_v)
        pltpu.async_copy(ys_hbm.at[idx_v], rows_v, sem).wait()
        pltpu.sync_copy(rows_v, g1_hbm.at[pl.ds(base, TPW)])
        pltpu.sync_copy(p2_hbm.at[pl.ds(base, TPW)], idx_v)
        pltpu.async_copy(ys_hbm.at[idx_v], rows_v, sem).wait()
        pltpu.sync_copy(rows_v, g2_hbm.at[pl.ds(base, TPW)])

    return body(ys, p1, p2)


def _shared_body(h_ref, wsg_ref, wsu_ref, wsd_ref, out_ref):
    hb = h_ref[...]
    a = jnp.dot(hb, wsg_ref[...], preferred_element_type=jnp.float32)
    u = jnp.dot(hb, wsu_ref[...], preferred_element_type=jnp.float32)
    out_ref[...] = jnp.dot(_silu(a) * u, wsd_ref[...],
                           preferred_element_type=jnp.float32)


def _shared_call(h, Ws_gate, Ws_up, Ws_down):
    BTOK = 512
    tok = lambda b: (b, 0)
    fixed = lambda b: (0, 0)
    return pl.pallas_call(
        _shared_body,
        grid=(T // BTOK,),
        in_specs=[
            pl.BlockSpec((BTOK, D), tok),
            pl.BlockSpec((D, IS), fixed),
            pl.BlockSpec((D, IS), fixed),
            pl.BlockSpec((IS, D), fixed),
        ],
        out_specs=pl.BlockSpec((BTOK, D), tok),
        out_shape=jax.ShapeDtypeStruct((T, D), jnp.float32),
    )(h, Ws_gate, Ws_up, Ws_down)


def _combine_body(x_ref, ysh_ref, meta_ref, g1_ref, g2_ref, out_ref):
    w1 = meta_ref[:, 2:3]
    w2 = meta_ref[:, 3:4]
    out_ref[...] = (x_ref[...] + ysh_ref[...]
                    + w1 * g1_ref[...] + w2 * g2_ref[...])


def _combine_call(x, ysh, meta, g1, g2):
    BTOK = 512
    tok = lambda b: (b, 0)
    return pl.pallas_call(
        _combine_body,
        grid=(T // BTOK,),
        in_specs=[
            pl.BlockSpec((BTOK, D), tok),
            pl.BlockSpec((BTOK, D), tok),
            pl.BlockSpec((BTOK, 8), tok),
            pl.BlockSpec((BTOK, D), tok),
            pl.BlockSpec((BTOK, D), tok),
        ],
        out_specs=pl.BlockSpec((BTOK, D), tok),
        out_shape=jax.ShapeDtypeStruct((T, D), jnp.float32),
    )(x, ysh, meta, g1, g2)


def kernel(x, norm_w, Wg_router, gate_bias, Ws_gate, Ws_up, Ws_down,
           We_gate, We_up, We_down):
    h, meta, smeta8 = _router_call(x, norm_w, Wg_router, gate_bias)
    p1 = meta[:, 0].astype(jnp.int32)
    p2 = meta[:, 1].astype(jnp.int32)
    smeta = smeta8[0]
    hs = _dispatch_sc(h, p1, p2)
    ysh = _shared_call(h, Ws_gate, Ws_up, Ws_down)
    ys = _ffn_call(smeta, hs, We_gate, We_up, We_down)
    g1, g2 = _gather_sc(ys, p1, p2)
    return _combine_call(x, ysh, meta, g1, g2)
